# baseline (speedup 1.0000x reference)
import functools

import jax
import jax.numpy as jnp
from jax import lax
from jax.experimental import pallas as pl
from jax.experimental.pallas import tpu as pltpu
from jax.experimental.pallas import tpu_sc as plsc

VOCAB = 1000000
DIM = 64
BATCH = 4096
HIST = 200
N = BATCH * HIST
NC, NS, L = 2, 16, 16
NW = NC * NS
PER_W = N // NW
C = 512
NCH = PER_W // C
IDX_ROWS = PER_W // 128


def _ln_body(x_hbm, tab_hbm, g_hbm, b_hbm, out_hbm,
             idx_v, rows_a, rows_b, g_v, b_v, stats_v,
             gs0, gs1, os0, os1):
    wid = lax.axis_index("s") * NC + lax.axis_index("c")
    base = wid * PER_W

    pltpu.sync_copy(x_hbm.at[pl.ds(wid * IDX_ROWS, IDX_ROWS)], idx_v)
    pltpu.sync_copy(g_hbm, g_v)
    pltpu.sync_copy(b_hbm, b_v)

    rows = (rows_a, rows_b)
    gsems = (gs0, gs1)
    osems = (os0, os1)

    def fire_gather(ch, slot):
        for j in range(4):
            pltpu.async_copy(tab_hbm.at[idx_v.at[ch * 4 + j]],
                             rows[slot].at[pl.ds(j * 128, 128)],
                             gsems[slot])

    def wait_gather(ch, slot):
        for j in range(4):
            pltpu.make_async_copy(tab_hbm.at[idx_v.at[ch * 4 + j]],
                                  rows[slot].at[pl.ds(j * 128, 128)],
                                  gsems[slot]).wait()

    def fire_write(ch, slot):
        pltpu.async_copy(rows[slot], out_hbm.at[pl.ds(base + ch * C, C)],
                         osems[slot])

    def wait_write(ch, slot):
        pltpu.make_async_copy(rows[slot], out_hbm.at[pl.ds(base + ch * C, C)],
                              osems[slot]).wait()

    def compute(slot):
        r_ref = rows[slot]

        def rg_body(rg, carry):
            r0 = rg * L
            lanes = r0 + lax.iota(jnp.int32, L)
            s = None
            q = None
            for j in range(DIM):
                cj = jnp.full((L,), j, jnp.int32)
                c = plsc.load_gather(r_ref, [lanes, cj])
                s = c if s is None else s + c
                q = c * c if q is None else q + c * c
            mean = s * (1.0 / DIM)
            var = q * (1.0 / DIM) - mean * mean
            xv = var + 1e-5
            ii = plsc.bitcast(xv, jnp.int32)
            ii = jnp.int32(0x5F3759DF) - lax.shift_right_logical(ii, 1)
            y = plsc.bitcast(ii, jnp.float32)
            hx = xv * 0.5
            for _ in range(3):
                y = y * (1.5 - hx * y * y)
            for rr in range(L):
                r = r0 + rr
                m_s = mean[rr]
                i_s = y[rr]
                for k in range(DIM // L):
                    v = r_ref[r, pl.ds(k * L, L)]
                    gk = g_v[pl.ds(k * L, L)]
                    bk = b_v[pl.ds(k * L, L)]
                    r_ref[r, pl.ds(k * L, L)] = (v - m_s) * i_s * gk + bk
            return carry

        lax.fori_loop(0, C // L, rg_body, 0)

    fire_gather(0, 0)

    def pair_body(i, carry):
        for b2 in (0, 1):
            ch = 2 * i + b2
            nxt = ch + 1

            @pl.when(nxt < NCH)
            def _():
                @pl.when(ch >= 1)
                def _():
                    wait_write(ch - 1, b2 ^ 1)
                fire_gather(nxt, b2 ^ 1)

            wait_gather(ch, b2)
            compute(b2)
            fire_write(ch, b2)
        return carry

    lax.fori_loop(0, NCH // 2, pair_body, 0)
    wait_write(NCH - 2, 0)
    wait_write(NCH - 1, 1)


_emb_ln = functools.partial(
    pl.kernel,
    out_type=jax.ShapeDtypeStruct((N, DIM), jnp.float32),
    mesh=plsc.VectorSubcoreMesh(core_axis_name="c", subcore_axis_name="s"),
    compiler_params=pltpu.CompilerParams(needs_layout_passes=False,
                                         use_tc_tiling_on_sc=False),
    scratch_types=[
        pltpu.VMEM((N // 128 // NW, 128), jnp.int32),
        pltpu.VMEM((C, DIM), jnp.float32),
        pltpu.VMEM((C, DIM), jnp.float32),
        pltpu.VMEM((DIM,), jnp.float32),
        pltpu.VMEM((DIM,), jnp.float32),
        pltpu.VMEM((2, L), jnp.float32),
        pltpu.SemaphoreType.DMA,
        pltpu.SemaphoreType.DMA,
        pltpu.SemaphoreType.DMA,
        pltpu.SemaphoreType.DMA,
    ],
)(_ln_body)


def kernel(x, table, gamma, beta):
    xf = x.reshape(N // 128, 128).astype(jnp.int32)
    out = _emb_ln(xf, table, gamma, beta)
    return out.reshape(BATCH, HIST, DIM)
